# baseline (speedup 1.0000x reference)
import dataclasses
import functools

import jax
import jax.numpy as jnp
from jax import lax
from jax.experimental import pallas as pl
from jax.experimental.pallas import tpu as pltpu
from jax.experimental.pallas import tpu_sc as plsc

N = 10000
E = 320000
DIN = 128
DOUT = 128

NC = 2
NS = 16
NW = NC * NS
CHUNK = 80
NCHUNK = 136
RCHUNK = 125
BLK = 8
NBLK = NCHUNK // BLK
NSELF = 320
NPADE = 560
ROWS_PER_SUB = 640
NPAD = NS * ROWS_PER_SUB

_mesh = plsc.VectorSubcoreMesh(core_axis_name="c", subcore_axis_name="s")

_cp = pltpu.CompilerParams()
if "needs_layout_passes" in pltpu.CompilerParams.__dataclass_fields__:
    _cp = dataclasses.replace(_cp, needs_layout_passes=False)


def _synth_tail(wid, idx_refs, ew_ref):
    iota = lax.iota(jnp.int32, 16)
    ones = jnp.ones((16,), jnp.float32)
    zeros = jnp.zeros((16,), jnp.float32)
    for r in range(RCHUNK, NCHUNK):
        for g in range(CHUNK // 16):
            j0 = (r - RCHUNK) * CHUNK + g * 16
            if j0 < NSELF:
                raw = wid * NSELF + j0 + iota
                w = jnp.where(raw < N, ones, zeros)
            else:
                raw = wid * NPADE + (j0 - NSELF) + iota
                w = zeros
            node = lax.rem(raw, N)
            sl = pl.ds(g * 16, 16)
            for ref in idx_refs:
                ref[r, sl] = node
            ew_ref[r, sl] = w


@functools.partial(
    pl.kernel,
    mesh=_mesh,
    out_type=jax.ShapeDtypeStruct((NC, NPAD), jnp.float32),
    compiler_params=_cp,
    scratch_types=[
        pltpu.VMEM((NCHUNK, CHUNK), jnp.int32),
        pltpu.VMEM((NCHUNK, CHUNK), jnp.float32),
        pltpu.VMEM((ROWS_PER_SUB,), jnp.float32),
        pltpu.VMEM_SHARED((NPAD,), jnp.float32),
    ],
)
def _deg_kernel(col_hbm, ew_hbm, deg_hbm, col_v, ew_v, z_v, acc):
    cid = lax.axis_index("c")
    sid = lax.axis_index("s")
    wid = cid * NS + sid

    pltpu.sync_copy(col_hbm.at[wid], col_v.at[pl.ds(0, RCHUNK)])
    pltpu.sync_copy(ew_hbm.at[wid], ew_v.at[pl.ds(0, RCHUNK)])
    _synth_tail(wid, [col_v], ew_v)

    @pl.loop(0, ROWS_PER_SUB // 16)
    def _(i):
        z_v[pl.ds(i * 16, 16)] = jnp.zeros((16,), jnp.float32)

    pltpu.sync_copy(z_v, acc.at[pl.ds(sid * ROWS_PER_SUB, ROWS_PER_SUB)])

    plsc.subcore_barrier()

    @pl.loop(0, NCHUNK)
    def _(c):
        pltpu.sync_copy(ew_v.at[c], acc.at[col_v.at[c]], add=True)

    plsc.subcore_barrier()

    pltpu.sync_copy(acc.at[pl.ds(sid * ROWS_PER_SUB, ROWS_PER_SUB)],
                    deg_hbm.at[cid, pl.ds(sid * ROWS_PER_SUB, ROWS_PER_SUB)])


def _mm_body(x_ref, w_ref, degp_ref, xw_ref, dinv_ref):
    xw_ref[...] = jnp.dot(x_ref[...], w_ref[...],
                          preferred_element_type=jnp.float32)
    deg = degp_ref[0:1, :] + degp_ref[1:2, :]
    dinv_ref[...] = lax.rsqrt(deg)


def _mm_dinv(x, W, degp):
    return pl.pallas_call(
        _mm_body,
        grid=(10,),
        in_specs=[
            pl.BlockSpec((1000, DIN), lambda i: (i, 0)),
            pl.BlockSpec((DIN, DOUT), lambda i: (0, 0)),
            pl.BlockSpec((NC, NPAD), lambda i: (0, 0)),
        ],
        out_specs=[
            pl.BlockSpec((1000, DOUT), lambda i: (i, 0)),
            pl.BlockSpec((1, NPAD), lambda i: (0, 0)),
        ],
        out_shape=[
            jax.ShapeDtypeStruct((N, DOUT), jnp.float32),
            jax.ShapeDtypeStruct((1, NPAD), jnp.float32),
        ],
    )(x, W, degp)


@functools.partial(
    pl.kernel,
    mesh=_mesh,
    out_type=[
        jax.ShapeDtypeStruct((NW, NCHUNK, CHUNK), jnp.float32),
        jax.ShapeDtypeStruct((NW, NCHUNK, CHUNK), jnp.int32),
        jax.ShapeDtypeStruct((NW, NCHUNK, CHUNK), jnp.int32),
    ],
    compiler_params=_cp,
    scratch_types=[
        pltpu.VMEM((NCHUNK, CHUNK), jnp.int32),
        pltpu.VMEM((NCHUNK, CHUNK), jnp.int32),
        pltpu.VMEM((NCHUNK, CHUNK), jnp.float32),
        pltpu.VMEM((NPAD,), jnp.float32),
        pltpu.VMEM((NCHUNK, CHUNK), jnp.float32),
    ],
)
def _sedge_kernel(dinv_hbm, row_hbm, col_hbm, ew_hbm,
                  s_hbm, rowp_hbm, colp_hbm,
                  row_v, col_v, ew_v, dinv_v, s_v):
    cid = lax.axis_index("c")
    sid = lax.axis_index("s")
    wid = cid * NS + sid

    pltpu.sync_copy(row_hbm.at[wid], row_v.at[pl.ds(0, RCHUNK)])
    pltpu.sync_copy(col_hbm.at[wid], col_v.at[pl.ds(0, RCHUNK)])
    pltpu.sync_copy(ew_hbm.at[wid], ew_v.at[pl.ds(0, RCHUNK)])
    pltpu.sync_copy(dinv_hbm.at[0], dinv_v)
    _synth_tail(wid, [row_v, col_v], ew_v)

    @pl.loop(0, NCHUNK)
    def _(c):
        for g in range(CHUNK // 16):
            sl = pl.ds(g * 16, 16)
            dr = plsc.load_gather(dinv_v, [row_v[c, sl]])
            dc = plsc.load_gather(dinv_v, [col_v[c, sl]])
            s_v[c, sl] = ew_v[c, sl] * dr * dc

    pltpu.sync_copy(s_v, s_hbm.at[wid])
    pltpu.sync_copy(row_v, rowp_hbm.at[wid])
    pltpu.sync_copy(col_v, colp_hbm.at[wid])


@functools.partial(
    pl.kernel,
    mesh=_mesh,
    out_type=jax.ShapeDtypeStruct((NC, NPAD, DOUT), jnp.float32),
    compiler_params=_cp,
    scratch_types=[
        pltpu.VMEM((2, BLK, 2, CHUNK // 2), jnp.int32),
        pltpu.VMEM((2, BLK, 2, CHUNK // 2), jnp.int32),
        pltpu.VMEM((2, BLK, CHUNK), jnp.float32),
        pltpu.VMEM((2, CHUNK, DOUT), jnp.float32),
        pltpu.VMEM((CHUNK,), jnp.float32),
        pltpu.SemaphoreType.DMA,
        pltpu.SemaphoreType.DMA,
        pltpu.SemaphoreType.DMA,
        pltpu.SemaphoreType.DMA,
        pltpu.SemaphoreType.DMA,
        pltpu.SemaphoreType.DMA,
        pltpu.SemaphoreType.DMA,
        pltpu.SemaphoreType.DMA,
        pltpu.VMEM_SHARED((NPAD, DOUT), jnp.float32),
    ],
)
def _scatter_kernel(xw_hbm, row_hbm, col_hbm, s_hbm, out_hbm,
                    rowb, colb, sb, rows2, s_v,
                    gsem00, gsem01, gsem10, gsem11,
                    ssem00, ssem01, ssem10, ssem11,
                    acc):
    cid = lax.axis_index("c")
    sid = lax.axis_index("s")
    wid = cid * NS + sid
    gsem = ((gsem00, gsem01), (gsem10, gsem11))
    ssem = ((ssem00, ssem01), (ssem10, ssem11))
    HC = CHUNK // 2

    def stage(blk_idx, slot):
        blk_sl = pl.ds(blk_idx * BLK, BLK)
        pltpu.sync_copy(row_hbm.at[wid, blk_sl], rowb.at[slot])
        pltpu.sync_copy(col_hbm.at[wid, blk_sl], colb.at[slot])
        pltpu.sync_copy(s_hbm.at[wid, blk_sl], sb.at[slot])

    def issue_gather(slot, j, p):
        for h in range(2):
            pltpu.async_copy(xw_hbm.at[rowb.at[slot].at[j].at[h]],
                             rows2.at[p].at[pl.ds(h * HC, HC)], gsem[p][h])

    def wait_gather(p):
        for h in range(2):
            pltpu.make_async_copy(xw_hbm.at[rowb.at[0].at[0].at[h]],
                                  rows2.at[p].at[pl.ds(h * HC, HC)],
                                  gsem[p][h]).wait()

    def issue_scatter(slot, j, p):
        for h in range(2):
            pltpu.async_copy(rows2.at[p].at[pl.ds(h * HC, HC)],
                             acc.at[colb.at[slot, j, h]], ssem[p][h],
                             add=True)

    def wait_scatter(p):
        for h in range(2):
            pltpu.make_async_copy(rows2.at[p].at[pl.ds(h * HC, HC)],
                                  acc.at[colb.at[0, 0, h]], ssem[p][h]).wait()

    @pl.loop(0, CHUNK)
    def _(i):
        for j in range(DOUT // 16):
            rows2[0, i, pl.ds(j * 16, 16)] = jnp.zeros((16,), jnp.float32)

    @pl.loop(0, ROWS_PER_SUB // CHUNK)
    def _(k):
        pltpu.sync_copy(rows2.at[0],
                        acc.at[pl.ds(sid * ROWS_PER_SUB + k * CHUNK, CHUNK)])

    plsc.subcore_barrier()

    stage(0, 0)
    issue_gather(0, 0, 0)

    @pl.loop(0, NBLK)
    def _(b):
        slot = b % 2
        nslot = 1 - slot

        @pl.when(b + 1 < NBLK)
        def _():
            stage(b + 1, nslot)

        for j in range(BLK):
            p = j % 2

            for g in range(CHUNK // 16):
                s_v[pl.ds(g * 16, 16)] = sb[slot, j, pl.ds(g * 16, 16)]

            wait_gather(p)

            if j + 1 < BLK:
                if j == 0:
                    @pl.when(b > 0)
                    def _():
                        wait_scatter(1 - p)
                else:
                    wait_scatter(1 - p)
                issue_gather(slot, j + 1, 1 - p)
            else:
                @pl.when(b + 1 < NBLK)
                def _():
                    wait_scatter(1 - p)
                    issue_gather(nslot, 0, 1 - p)

            @pl.loop(0, CHUNK, step=2)
            def _(i):
                sv0 = plsc.load_gather(s_v, [jnp.full((16,), i, jnp.int32)])
                sv1 = plsc.load_gather(s_v, [jnp.full((16,), i + 1, jnp.int32)])
                for jj in range(DOUT // 16):
                    sl = pl.ds(jj * 16, 16)
                    rows2[p, i, sl] = rows2[p, i, sl] * sv0
                    rows2[p, i + 1, sl] = rows2[p, i + 1, sl] * sv1

            issue_scatter(slot, j, p)

    wait_scatter(0)
    wait_scatter(1)

    plsc.subcore_barrier()

    pltpu.sync_copy(acc.at[pl.ds(sid * ROWS_PER_SUB, ROWS_PER_SUB)],
                    out_hbm.at[cid, pl.ds(sid * ROWS_PER_SUB, ROWS_PER_SUB)])


def _fin_body(p_ref, b_ref, o_ref):
    o_ref[...] = p_ref[0] + p_ref[1] + b_ref[...]


def _final(partials, b):
    return pl.pallas_call(
        _fin_body,
        grid=(10,),
        in_specs=[
            pl.BlockSpec((NC, 1000, DOUT), lambda i: (0, i, 0)),
            pl.BlockSpec((1, DOUT), lambda i: (0, 0)),
        ],
        out_specs=pl.BlockSpec((1000, DOUT), lambda i: (i, 0)),
        out_shape=jax.ShapeDtypeStruct((N, DOUT), jnp.float32),
    )(partials, b.reshape(1, DOUT))


@jax.jit
def kernel(x, edge_index, edge_attr, W, b):
    row3 = edge_index[0].reshape(NW, RCHUNK, CHUNK)
    col3 = edge_index[1].reshape(NW, RCHUNK, CHUNK)
    ew3 = edge_attr.reshape(NW, RCHUNK, CHUNK)

    degp = _deg_kernel(col3, ew3)
    xw, dinv = _mm_dinv(x, W, degp)
    s3, row3p, col3p = _sedge_kernel(dinv, row3, col3, ew3)
    row4p = row3p.reshape(NW, NCHUNK, 2, CHUNK // 2)
    col4p = col3p.reshape(NW, NCHUNK, 2, CHUNK // 2)
    partials = _scatter_kernel(xw, row4p, col4p, s3)
    out = _final(partials, b)
    return (out, edge_index, edge_attr)


# speedup vs baseline: 1.0317x; 1.0317x over previous
import dataclasses
import functools

import jax
import jax.numpy as jnp
from jax import lax
from jax.experimental import pallas as pl
from jax.experimental.pallas import tpu as pltpu
from jax.experimental.pallas import tpu_sc as plsc

N = 10000
E = 320000
DIN = 128
DOUT = 128

NC = 2
NS = 16
NW = NC * NS
CHUNK = 80
NCHUNK = 136
RCHUNK = 125
BLK = 8
NBLK = NCHUNK // BLK
NSELF = 320
NPADE = 560
ROWS_PER_SUB = 640
NPAD = NS * ROWS_PER_SUB

_mesh = plsc.VectorSubcoreMesh(core_axis_name="c", subcore_axis_name="s")

_cp = pltpu.CompilerParams()
if "needs_layout_passes" in pltpu.CompilerParams.__dataclass_fields__:
    _cp = dataclasses.replace(_cp, needs_layout_passes=False)


def _synth_tail(wid, idx_refs, ew_ref):
    iota = lax.iota(jnp.int32, 16)
    ones = jnp.ones((16,), jnp.float32)
    zeros = jnp.zeros((16,), jnp.float32)
    for r in range(RCHUNK, NCHUNK):
        for g in range(CHUNK // 16):
            j0 = (r - RCHUNK) * CHUNK + g * 16
            if j0 < NSELF:
                raw = wid * NSELF + j0 + iota
                w = jnp.where(raw < N, ones, zeros)
            else:
                raw = wid * NPADE + (j0 - NSELF) + iota
                w = zeros
            node = lax.rem(raw, N)
            sl = pl.ds(g * 16, 16)
            for ref in idx_refs:
                ref[r, sl] = node
            ew_ref[r, sl] = w


@functools.partial(
    pl.kernel,
    mesh=_mesh,
    out_type=jax.ShapeDtypeStruct((NC, NPAD), jnp.float32),
    compiler_params=_cp,
    scratch_types=[
        pltpu.VMEM((NCHUNK, CHUNK), jnp.int32),
        pltpu.VMEM((NCHUNK, CHUNK), jnp.float32),
        pltpu.VMEM((ROWS_PER_SUB,), jnp.float32),
        pltpu.VMEM_SHARED((NPAD,), jnp.float32),
    ],
)
def _deg_kernel(col_hbm, ew_hbm, deg_hbm, col_v, ew_v, z_v, acc):
    cid = lax.axis_index("c")
    sid = lax.axis_index("s")
    wid = cid * NS + sid

    pltpu.sync_copy(col_hbm.at[wid], col_v.at[pl.ds(0, RCHUNK)])
    pltpu.sync_copy(ew_hbm.at[wid], ew_v.at[pl.ds(0, RCHUNK)])
    _synth_tail(wid, [col_v], ew_v)

    @pl.loop(0, ROWS_PER_SUB // 16)
    def _(i):
        z_v[pl.ds(i * 16, 16)] = jnp.zeros((16,), jnp.float32)

    pltpu.sync_copy(z_v, acc.at[pl.ds(sid * ROWS_PER_SUB, ROWS_PER_SUB)])

    plsc.subcore_barrier()

    @pl.loop(0, NCHUNK)
    def _(c):
        pltpu.sync_copy(ew_v.at[c], acc.at[col_v.at[c]], add=True)

    plsc.subcore_barrier()

    pltpu.sync_copy(acc.at[pl.ds(sid * ROWS_PER_SUB, ROWS_PER_SUB)],
                    deg_hbm.at[cid, pl.ds(sid * ROWS_PER_SUB, ROWS_PER_SUB)])


def _mm_body(x_ref, w_ref, degp_ref, xw_ref, dinv_ref):
    xw_ref[...] = jnp.dot(x_ref[...], w_ref[...],
                          preferred_element_type=jnp.float32)
    deg = degp_ref[0:1, :] + degp_ref[1:2, :]
    dinv_ref[...] = lax.rsqrt(deg)


def _mm_dinv(x, W, degp):
    return pl.pallas_call(
        _mm_body,
        grid=(10,),
        in_specs=[
            pl.BlockSpec((1000, DIN), lambda i: (i, 0)),
            pl.BlockSpec((DIN, DOUT), lambda i: (0, 0)),
            pl.BlockSpec((NC, NPAD), lambda i: (0, 0)),
        ],
        out_specs=[
            pl.BlockSpec((1000, DOUT), lambda i: (i, 0)),
            pl.BlockSpec((1, NPAD), lambda i: (0, 0)),
        ],
        out_shape=[
            jax.ShapeDtypeStruct((N, DOUT), jnp.float32),
            jax.ShapeDtypeStruct((1, NPAD), jnp.float32),
        ],
    )(x, W, degp)


@functools.partial(
    pl.kernel,
    mesh=_mesh,
    out_type=[
        jax.ShapeDtypeStruct((NW, NCHUNK, CHUNK), jnp.float32),
        jax.ShapeDtypeStruct((NW, NCHUNK, CHUNK), jnp.int32),
        jax.ShapeDtypeStruct((NW, NCHUNK, CHUNK), jnp.int32),
    ],
    compiler_params=_cp,
    scratch_types=[
        pltpu.VMEM((NCHUNK, CHUNK), jnp.int32),
        pltpu.VMEM((NCHUNK, CHUNK), jnp.int32),
        pltpu.VMEM((NCHUNK, CHUNK), jnp.float32),
        pltpu.VMEM((NPAD,), jnp.float32),
        pltpu.VMEM((NCHUNK, CHUNK), jnp.float32),
    ],
)
def _sedge_kernel(dinv_hbm, row_hbm, col_hbm, ew_hbm,
                  s_hbm, rowp_hbm, colp_hbm,
                  row_v, col_v, ew_v, dinv_v, s_v):
    cid = lax.axis_index("c")
    sid = lax.axis_index("s")
    wid = cid * NS + sid

    pltpu.sync_copy(row_hbm.at[wid], row_v.at[pl.ds(0, RCHUNK)])
    pltpu.sync_copy(col_hbm.at[wid], col_v.at[pl.ds(0, RCHUNK)])
    pltpu.sync_copy(ew_hbm.at[wid], ew_v.at[pl.ds(0, RCHUNK)])
    pltpu.sync_copy(dinv_hbm.at[0], dinv_v)
    _synth_tail(wid, [row_v, col_v], ew_v)

    @pl.loop(0, NCHUNK)
    def _(c):
        for g in range(CHUNK // 16):
            sl = pl.ds(g * 16, 16)
            dr = plsc.load_gather(dinv_v, [row_v[c, sl]])
            dc = plsc.load_gather(dinv_v, [col_v[c, sl]])
            s_v[c, sl] = ew_v[c, sl] * dr * dc

    pltpu.sync_copy(s_v, s_hbm.at[wid])
    pltpu.sync_copy(row_v, rowp_hbm.at[wid])
    pltpu.sync_copy(col_v, colp_hbm.at[wid])


@functools.partial(
    pl.kernel,
    mesh=_mesh,
    out_type=jax.ShapeDtypeStruct((NC, NPAD, DOUT), jnp.float32),
    compiler_params=_cp,
    scratch_types=[
        pltpu.VMEM((2, BLK, CHUNK), jnp.int32),
        pltpu.VMEM((2, BLK, CHUNK), jnp.int32),
        pltpu.VMEM((2, BLK, CHUNK), jnp.float32),
        pltpu.VMEM((2, CHUNK, DOUT), jnp.float32),
        pltpu.VMEM((CHUNK,), jnp.float32),
        pltpu.SemaphoreType.DMA,
        pltpu.SemaphoreType.DMA,
        pltpu.SemaphoreType.DMA,
        pltpu.SemaphoreType.DMA,
        pltpu.VMEM_SHARED((NPAD, DOUT), jnp.float32),
    ],
)
def _scatter_kernel(xw_hbm, row_hbm, col_hbm, s_hbm, out_hbm,
                    rowb, colb, sb, rows2, s_v, gsem0, gsem1, ssem0, ssem1,
                    acc):
    cid = lax.axis_index("c")
    sid = lax.axis_index("s")
    wid = cid * NS + sid
    gsem = (gsem0, gsem1)
    ssem = (ssem0, ssem1)

    def stage(blk_idx, slot):
        blk_sl = pl.ds(blk_idx * BLK, BLK)
        pltpu.sync_copy(row_hbm.at[wid, blk_sl], rowb.at[slot])
        pltpu.sync_copy(col_hbm.at[wid, blk_sl], colb.at[slot])
        pltpu.sync_copy(s_hbm.at[wid, blk_sl], sb.at[slot])

    def issue_gather(slot, j, p):
        pltpu.async_copy(xw_hbm.at[rowb.at[slot].at[j]], rows2.at[p], gsem[p])

    def wait_gather(p):
        pltpu.make_async_copy(xw_hbm.at[rowb.at[0].at[0]], rows2.at[p],
                              gsem[p]).wait()

    def issue_scatter(slot, j, p):
        pltpu.async_copy(rows2.at[p], acc.at[colb.at[slot].at[j]], ssem[p],
                         add=True)

    def wait_scatter(p):
        pltpu.make_async_copy(rows2.at[p], acc.at[colb.at[0].at[0]],
                              ssem[p]).wait()

    @pl.loop(0, CHUNK)
    def _(i):
        for j in range(DOUT // 16):
            rows2[0, i, pl.ds(j * 16, 16)] = jnp.zeros((16,), jnp.float32)

    @pl.loop(0, ROWS_PER_SUB // CHUNK)
    def _(k):
        pltpu.sync_copy(rows2.at[0],
                        acc.at[pl.ds(sid * ROWS_PER_SUB + k * CHUNK, CHUNK)])

    plsc.subcore_barrier()

    stage(0, 0)
    issue_gather(0, 0, 0)

    @pl.loop(0, NBLK)
    def _(b):
        slot = b % 2
        nslot = 1 - slot

        @pl.when(b + 1 < NBLK)
        def _():
            stage(b + 1, nslot)

        for j in range(BLK):
            p = j % 2

            for g in range(CHUNK // 16):
                s_v[pl.ds(g * 16, 16)] = sb[slot, j, pl.ds(g * 16, 16)]

            wait_gather(p)

            if j + 1 < BLK:
                if j == 0:
                    @pl.when(b > 0)
                    def _():
                        wait_scatter(1 - p)
                else:
                    wait_scatter(1 - p)
                issue_gather(slot, j + 1, 1 - p)
            else:
                @pl.when(b + 1 < NBLK)
                def _():
                    wait_scatter(1 - p)
                    issue_gather(nslot, 0, 1 - p)

            @pl.loop(0, CHUNK, step=2)
            def _(i):
                sv0 = plsc.load_gather(s_v, [jnp.full((16,), i, jnp.int32)])
                sv1 = plsc.load_gather(s_v, [jnp.full((16,), i + 1, jnp.int32)])
                for jj in range(DOUT // 16):
                    sl = pl.ds(jj * 16, 16)
                    rows2[p, i, sl] = rows2[p, i, sl] * sv0
                    rows2[p, i + 1, sl] = rows2[p, i + 1, sl] * sv1

            issue_scatter(slot, j, p)

    wait_scatter(0)
    wait_scatter(1)

    plsc.subcore_barrier()

    pltpu.sync_copy(acc.at[pl.ds(sid * ROWS_PER_SUB, ROWS_PER_SUB)],
                    out_hbm.at[cid, pl.ds(sid * ROWS_PER_SUB, ROWS_PER_SUB)])


def _fin_body(p_ref, b_ref, o_ref):
    o_ref[...] = p_ref[0] + p_ref[1] + b_ref[...]


def _final(partials, b):
    return pl.pallas_call(
        _fin_body,
        grid=(10,),
        in_specs=[
            pl.BlockSpec((NC, 1000, DOUT), lambda i: (0, i, 0)),
            pl.BlockSpec((1, DOUT), lambda i: (0, 0)),
        ],
        out_specs=pl.BlockSpec((1000, DOUT), lambda i: (i, 0)),
        out_shape=jax.ShapeDtypeStruct((N, DOUT), jnp.float32),
    )(partials, b.reshape(1, DOUT))


@jax.jit
def kernel(x, edge_index, edge_attr, W, b):
    row3 = edge_index[0].reshape(NW, RCHUNK, CHUNK)
    col3 = edge_index[1].reshape(NW, RCHUNK, CHUNK)
    ew3 = edge_attr.reshape(NW, RCHUNK, CHUNK)

    degp = _deg_kernel(col3, ew3)
    xw, dinv = _mm_dinv(x, W, degp)
    s3, row3p, col3p = _sedge_kernel(dinv, row3, col3, ew3)
    partials = _scatter_kernel(xw, row3p, col3p, s3)
    out = _final(partials, b)
    return (out, edge_index, edge_attr)
